# Initial kernel scaffold; baseline (speedup 1.0000x reference)
import jax
import jax.numpy as jnp
from jax.experimental import pallas as pl


def kernel(x, edge_index, Wl0, bl0, Wr0, Wl1, bl1, Wr1, Wl2, bl2, Wr2):
    raise NotImplementedError("write your pallas kernel here")


# speedup vs baseline: 7.6462x; 7.6462x over previous
import functools

import jax
import jax.numpy as jnp
from jax import lax
from jax.experimental import pallas as pl
from jax.experimental.pallas import tpu as pltpu
from jax.experimental.pallas import tpu_sc as plsc

N = 10000
NP = 10240
E = 320000
D = 128
NC = 2
NS = 16
NW = NC * NS
EPW = E // NW
CH = 125
NCHUNK = EPW // CH
RPT = NP // NS

@functools.cache
def _make_agg():
  mesh = plsc.VectorSubcoreMesh(core_axis_name="c", subcore_axis_name="s")

  @functools.partial(
      pl.kernel,
      out_type=jax.ShapeDtypeStruct((NC, NP, D), jnp.float32),
      mesh=mesh,
      scratch_types=[
          pltpu.VMEM((NCHUNK, CH), jnp.int32),
          pltpu.VMEM((NCHUNK, CH), jnp.int32),
          pltpu.VMEM((CH, D), jnp.float32),
          pltpu.VMEM_SHARED((NP, D), jnp.float32),
          pltpu.SemaphoreType.DMA,
      ],
  )
  def _agg(x_hbm, src_hbm, dst_hbm, zero_hbm, out_hbm, src_v, dst_v, rows_v,
           acc_sh, sem):
    c = lax.axis_index("c")
    s = lax.axis_index("s")
    w = c * NS + s
    pltpu.sync_copy(src_hbm.at[w], src_v)
    pltpu.sync_copy(dst_hbm.at[w], dst_v)
    pltpu.sync_copy(zero_hbm.at[pl.ds(s * RPT, RPT)],
                    acc_sh.at[pl.ds(s * RPT, RPT)])
    plsc.subcore_barrier()

    def body(j, carry):
      pltpu.async_copy(x_hbm.at[src_v.at[j]], rows_v, sem).wait()
      pltpu.sync_copy(rows_v, acc_sh.at[dst_v.at[j]], add=True)
      return carry

    lax.fori_loop(0, NCHUNK, body, 0)
    plsc.subcore_barrier()
    pltpu.sync_copy(acc_sh.at[pl.ds(s * RPT, RPT)],
                    out_hbm.at[c].at[pl.ds(s * RPT, RPT)])

  return _agg


@functools.cache
def _make_cnt():
  mesh = plsc.VectorSubcoreMesh(core_axis_name="c", subcore_axis_name="s")

  @functools.partial(
      pl.kernel,
      out_type=jax.ShapeDtypeStruct((NC, NP, D), jnp.float32),
      mesh=mesh,
      scratch_types=[
          pltpu.VMEM((NCHUNK, CH), jnp.int32),
          pltpu.VMEM((CH, D), jnp.float32),
          pltpu.VMEM_SHARED((NP, D), jnp.float32),
      ],
  )
  def _cnt(dst_hbm, ones_hbm, zero_hbm, out_hbm, dst_v, ones_v, acc_sh):
    c = lax.axis_index("c")
    s = lax.axis_index("s")
    w = c * NS + s
    pltpu.sync_copy(dst_hbm.at[w], dst_v)
    pltpu.sync_copy(ones_hbm, ones_v)
    pltpu.sync_copy(zero_hbm.at[pl.ds(s * RPT, RPT)],
                    acc_sh.at[pl.ds(s * RPT, RPT)])
    plsc.subcore_barrier()

    def body(j, carry):
      pltpu.sync_copy(ones_v, acc_sh.at[dst_v.at[j]], add=True)
      return carry

    lax.fori_loop(0, NCHUNK, body, 0)
    plsc.subcore_barrier()
    pltpu.sync_copy(acc_sh.at[pl.ds(s * RPT, RPT)],
                    out_hbm.at[c].at[pl.ds(s * RPT, RPT)])

  return _cnt


BT = 1024


def _inv_body(c0_ref, c1_ref, o_ref):
    o_ref[...] = 1.0 / jnp.maximum(c0_ref[...] + c1_ref[...], 1.0)


def _inv(c0, c1):
    return pl.pallas_call(
        _inv_body,
        out_shape=jax.ShapeDtypeStruct((NP, D), jnp.float32),
        grid=(NP // BT,),
        in_specs=[
            pl.BlockSpec((BT, D), lambda i: (i, 0)),
            pl.BlockSpec((BT, D), lambda i: (i, 0)),
        ],
        out_specs=pl.BlockSpec((BT, D), lambda i: (i, 0)),
    )(c0, c1)


def _layer_body(p0_ref, p1_ref, inv_ref, x_ref, wl_ref, bl_ref, wr_ref, o_ref,
                *, relu):
    mean = (p0_ref[...] + p1_ref[...]) * inv_ref[...]
    h = jnp.dot(mean, wl_ref[...], preferred_element_type=jnp.float32)
    h = h + bl_ref[...]
    h = h + jnp.dot(x_ref[...], wr_ref[...], preferred_element_type=jnp.float32)
    if relu:
        h = jnp.maximum(h, 0.0)
    o_ref[...] = h


def _layer(p0, p1, inv, x, Wl, bl, Wr, relu):
    return pl.pallas_call(
        functools.partial(_layer_body, relu=relu),
        out_shape=jax.ShapeDtypeStruct((NP, D), jnp.float32),
        grid=(NP // BT,),
        in_specs=[
            pl.BlockSpec((BT, D), lambda i: (i, 0)),
            pl.BlockSpec((BT, D), lambda i: (i, 0)),
            pl.BlockSpec((BT, D), lambda i: (i, 0)),
            pl.BlockSpec((BT, D), lambda i: (i, 0)),
            pl.BlockSpec((D, D), lambda i: (0, 0)),
            pl.BlockSpec((1, D), lambda i: (0, 0)),
            pl.BlockSpec((D, D), lambda i: (0, 0)),
        ],
        out_specs=pl.BlockSpec((BT, D), lambda i: (i, 0)),
    )(p0, p1, inv, x, Wl, bl.reshape(1, D), Wr)


def kernel(x, edge_index, Wl0, bl0, Wr0, Wl1, bl1, Wr1, Wl2, bl2, Wr2):
    src = edge_index[0].reshape(NW, NCHUNK, CH)
    dst = edge_index[1].reshape(NW, NCHUNK, CH)
    zeros = jnp.zeros((NP, D), jnp.float32)
    ones = jnp.ones((CH, D), jnp.float32)

    cnt = _make_cnt()(dst, ones, zeros)
    inv = _inv(cnt[0], cnt[1])

    agg = _make_agg()
    h = jnp.pad(x, ((0, NP - N), (0, 0)))
    for Wl, bl, Wr, relu in ((Wl0, bl0, Wr0, True),
                             (Wl1, bl1, Wr1, True),
                             (Wl2, bl2, Wr2, False)):
        p = agg(h, src, dst, zeros)
        h = _layer(p[0], p[1], inv, h, Wl, bl, Wr, relu)
    return h[:N]


# speedup vs baseline: 8.2993x; 1.0854x over previous
import functools

import jax
import jax.numpy as jnp
from jax import lax
from jax.experimental import pallas as pl
from jax.experimental.pallas import tpu as pltpu
from jax.experimental.pallas import tpu_sc as plsc

N = 10000
NP = 10240
E = 320000
D = 128
NC = 2
NS = 16
NW = NC * NS
EPW = E // NW
CH = 80
NCHUNK = EPW // CH
RPT = NP // NS

@functools.cache
def _make_agg():
  mesh = plsc.VectorSubcoreMesh(core_axis_name="c", subcore_axis_name="s")

  @functools.partial(
      pl.kernel,
      out_type=jax.ShapeDtypeStruct((NC, NP, D), jnp.float32),
      mesh=mesh,
      scratch_types=[
          pltpu.VMEM((EPW,), jnp.int32),
          pltpu.VMEM((NCHUNK, CH), jnp.int32),
          pltpu.VMEM((CH, D), jnp.float32),
          pltpu.VMEM((CH, D), jnp.float32),
          pltpu.VMEM_SHARED((NP, D), jnp.float32),
          pltpu.SemaphoreType.DMA,
          pltpu.SemaphoreType.DMA,
      ],
  )
  def _agg(x_hbm, src_hbm, dst_hbm, zero_hbm, out_hbm, src_v, dst_v, rows_a,
           rows_b, acc_sh, sem_a, sem_b):
    c = lax.axis_index("c")
    s = lax.axis_index("s")
    w = c * NS + s
    pltpu.sync_copy(src_hbm.at[w], src_v)
    pltpu.sync_copy(dst_hbm.at[w], dst_v)
    pltpu.sync_copy(zero_hbm.at[pl.ds(s * RPT, RPT)],
                    acc_sh.at[pl.ds(s * RPT, RPT)])
    plsc.subcore_barrier()

    pltpu.async_copy(x_hbm.at[src_v.at[pl.ds(0, CH)]], rows_a, sem_a)

    def body(i, carry):
      j0 = 2 * i
      pltpu.make_async_copy(x_hbm.at[src_v.at[pl.ds(j0 * CH, CH)]], rows_a,
                            sem_a).wait()
      pltpu.async_copy(x_hbm.at[src_v.at[pl.ds((j0 + 1) * CH, CH)]], rows_b,
                       sem_b)
      pltpu.sync_copy(rows_a, acc_sh.at[dst_v.at[j0]], add=True)
      pltpu.make_async_copy(x_hbm.at[src_v.at[pl.ds((j0 + 1) * CH, CH)]],
                            rows_b, sem_b).wait()

      @pl.when(j0 + 2 < NCHUNK)
      def _():
        pltpu.async_copy(x_hbm.at[src_v.at[pl.ds((j0 + 2) * CH, CH)]], rows_a,
                         sem_a)

      pltpu.sync_copy(rows_b, acc_sh.at[dst_v.at[j0 + 1]], add=True)
      return carry

    lax.fori_loop(0, NCHUNK // 2, body, 0)
    pltpu.make_async_copy(x_hbm.at[src_v.at[pl.ds((NCHUNK - 1) * CH, CH)]],
                          rows_a, sem_a).wait()
    pltpu.sync_copy(rows_a, acc_sh.at[dst_v.at[NCHUNK - 1]], add=True)
    plsc.subcore_barrier()
    pltpu.sync_copy(acc_sh.at[pl.ds(s * RPT, RPT)],
                    out_hbm.at[c].at[pl.ds(s * RPT, RPT)])

  return _agg


@functools.cache
def _make_cnt():
  mesh = plsc.VectorSubcoreMesh(core_axis_name="c", subcore_axis_name="s")

  @functools.partial(
      pl.kernel,
      out_type=jax.ShapeDtypeStruct((NC, NP, D), jnp.float32),
      mesh=mesh,
      scratch_types=[
          pltpu.VMEM((NCHUNK, CH), jnp.int32),
          pltpu.VMEM((CH, D), jnp.float32),
          pltpu.VMEM_SHARED((NP, D), jnp.float32),
      ],
  )
  def _cnt(dst_hbm, ones_hbm, zero_hbm, out_hbm, dst_v, ones_v, acc_sh):
    c = lax.axis_index("c")
    s = lax.axis_index("s")
    w = c * NS + s
    pltpu.sync_copy(dst_hbm.at[w], dst_v)
    pltpu.sync_copy(ones_hbm, ones_v)
    pltpu.sync_copy(zero_hbm.at[pl.ds(s * RPT, RPT)],
                    acc_sh.at[pl.ds(s * RPT, RPT)])
    plsc.subcore_barrier()

    def body(j, carry):
      pltpu.sync_copy(ones_v, acc_sh.at[dst_v.at[j]], add=True)
      return carry

    lax.fori_loop(0, NCHUNK, body, 0)
    plsc.subcore_barrier()
    pltpu.sync_copy(acc_sh.at[pl.ds(s * RPT, RPT)],
                    out_hbm.at[c].at[pl.ds(s * RPT, RPT)])

  return _cnt


BT = 1024


def _inv_body(c0_ref, c1_ref, o_ref):
    o_ref[...] = 1.0 / jnp.maximum(c0_ref[...] + c1_ref[...], 1.0)


def _inv(c0, c1):
    return pl.pallas_call(
        _inv_body,
        out_shape=jax.ShapeDtypeStruct((NP, D), jnp.float32),
        grid=(NP // BT,),
        in_specs=[
            pl.BlockSpec((BT, D), lambda i: (i, 0)),
            pl.BlockSpec((BT, D), lambda i: (i, 0)),
        ],
        out_specs=pl.BlockSpec((BT, D), lambda i: (i, 0)),
    )(c0, c1)


def _layer_body(p0_ref, p1_ref, inv_ref, x_ref, wl_ref, bl_ref, wr_ref, o_ref,
                *, relu):
    mean = (p0_ref[...] + p1_ref[...]) * inv_ref[...]
    h = jnp.dot(mean, wl_ref[...], preferred_element_type=jnp.float32)
    h = h + bl_ref[...]
    h = h + jnp.dot(x_ref[...], wr_ref[...], preferred_element_type=jnp.float32)
    if relu:
        h = jnp.maximum(h, 0.0)
    o_ref[...] = h


def _layer(p0, p1, inv, x, Wl, bl, Wr, relu):
    return pl.pallas_call(
        functools.partial(_layer_body, relu=relu),
        out_shape=jax.ShapeDtypeStruct((NP, D), jnp.float32),
        grid=(NP // BT,),
        in_specs=[
            pl.BlockSpec((BT, D), lambda i: (i, 0)),
            pl.BlockSpec((BT, D), lambda i: (i, 0)),
            pl.BlockSpec((BT, D), lambda i: (i, 0)),
            pl.BlockSpec((BT, D), lambda i: (i, 0)),
            pl.BlockSpec((D, D), lambda i: (0, 0)),
            pl.BlockSpec((1, D), lambda i: (0, 0)),
            pl.BlockSpec((D, D), lambda i: (0, 0)),
        ],
        out_specs=pl.BlockSpec((BT, D), lambda i: (i, 0)),
    )(p0, p1, inv, x, Wl, bl.reshape(1, D), Wr)


def kernel(x, edge_index, Wl0, bl0, Wr0, Wl1, bl1, Wr1, Wl2, bl2, Wr2):
    src = edge_index[0].reshape(NW, EPW)
    dst = edge_index[1].reshape(NW, NCHUNK, CH)
    zeros = jnp.zeros((NP, D), jnp.float32)
    ones = jnp.ones((CH, D), jnp.float32)

    cnt = _make_cnt()(dst, ones, zeros)
    inv = _inv(cnt[0], cnt[1])

    agg = _make_agg()
    h = jnp.pad(x, ((0, NP - N), (0, 0)))
    for Wl, bl, Wr, relu in ((Wl0, bl0, Wr0, True),
                             (Wl1, bl1, Wr1, True),
                             (Wl2, bl2, Wr2, False)):
        p = agg(h, src, dst, zeros)
        h = _layer(p[0], p[1], inv, h, Wl, bl, Wr, relu)
    return h[:N]


# speedup vs baseline: 8.3846x; 1.0103x over previous
import functools

import jax
import jax.numpy as jnp
from jax import lax
from jax.experimental import pallas as pl
from jax.experimental.pallas import tpu as pltpu
from jax.experimental.pallas import tpu_sc as plsc

N = 10000
NP = 10240
E = 320000
D = 128
NC = 2
NS = 16
NW = NC * NS
EPW = E // NW
CH = 80
NCHUNK = EPW // CH
RPT = NP // NS

@functools.cache
def _make_agg():
  mesh = plsc.VectorSubcoreMesh(core_axis_name="c", subcore_axis_name="s")

  @functools.partial(
      pl.kernel,
      out_type=jax.ShapeDtypeStruct((NC, NP, D), jnp.float32),
      mesh=mesh,
      scratch_types=[
          pltpu.VMEM((EPW,), jnp.int32),
          pltpu.VMEM((NCHUNK, CH), jnp.int32),
          pltpu.VMEM((CH, D), jnp.float32),
          pltpu.VMEM((CH, D), jnp.float32),
          pltpu.VMEM_SHARED((NP, D), jnp.float32),
          pltpu.SemaphoreType.DMA,
          pltpu.SemaphoreType.DMA,
          pltpu.SemaphoreType.DMA,
          pltpu.SemaphoreType.DMA,
      ],
  )
  def _agg(x_hbm, src_hbm, dst_hbm, zero_hbm, out_hbm, src_v, dst_v, rows_a,
           rows_b, acc_sh, sem_ga, sem_gb, sem_sa, sem_sb):
    c = lax.axis_index("c")
    s = lax.axis_index("s")
    w = c * NS + s
    pltpu.sync_copy(src_hbm.at[w], src_v)
    pltpu.sync_copy(dst_hbm.at[w], dst_v)
    pltpu.sync_copy(zero_hbm.at[pl.ds(s * RPT, RPT)],
                    acc_sh.at[pl.ds(s * RPT, RPT)])
    plsc.subcore_barrier()

    HC = CH // 2

    def fire_gather(j, buf, sem):
      pltpu.async_copy(x_hbm.at[src_v.at[pl.ds(j * CH, HC)]],
                       buf.at[pl.ds(0, HC)], sem)
      pltpu.async_copy(x_hbm.at[src_v.at[pl.ds(j * CH + HC, HC)]],
                       buf.at[pl.ds(HC, HC)], sem)

    def wait_gather(j, buf, sem):
      pltpu.make_async_copy(x_hbm.at[src_v.at[pl.ds(j * CH, HC)]],
                            buf.at[pl.ds(0, HC)], sem).wait()
      pltpu.make_async_copy(x_hbm.at[src_v.at[pl.ds(j * CH + HC, HC)]],
                            buf.at[pl.ds(HC, HC)], sem).wait()

    fire_gather(0, rows_a, sem_ga)
    fire_gather(1, rows_b, sem_gb)

    def body(i, carry):
      j0 = 2 * i
      wait_gather(j0, rows_a, sem_ga)
      pltpu.async_copy(rows_a, acc_sh.at[dst_v.at[j0]], sem_sa, add=True)
      wait_gather(j0 + 1, rows_b, sem_gb)
      pltpu.async_copy(rows_b, acc_sh.at[dst_v.at[j0 + 1]], sem_sb, add=True)

      @pl.when(j0 + 2 < NCHUNK)
      def _():
        pltpu.make_async_copy(rows_a, acc_sh.at[dst_v.at[j0]], sem_sa).wait()
        fire_gather(j0 + 2, rows_a, sem_ga)

      @pl.when(j0 + 3 < NCHUNK)
      def _():
        pltpu.make_async_copy(rows_b, acc_sh.at[dst_v.at[j0 + 1]],
                              sem_sb).wait()
        fire_gather(j0 + 3, rows_b, sem_gb)

      return carry

    lax.fori_loop(0, NCHUNK // 2, body, 0)
    wait_gather(NCHUNK - 1, rows_a, sem_ga)
    pltpu.async_copy(rows_a, acc_sh.at[dst_v.at[NCHUNK - 1]], sem_sa, add=True)
    pltpu.make_async_copy(rows_a, acc_sh.at[dst_v.at[NCHUNK - 1]],
                          sem_sa).wait()
    pltpu.make_async_copy(rows_b, acc_sh.at[dst_v.at[NCHUNK - 2]],
                          sem_sb).wait()
    plsc.subcore_barrier()
    pltpu.sync_copy(acc_sh.at[pl.ds(s * RPT, RPT)],
                    out_hbm.at[c].at[pl.ds(s * RPT, RPT)])

  return _agg


@functools.cache
def _make_cnt():
  mesh = plsc.VectorSubcoreMesh(core_axis_name="c", subcore_axis_name="s")

  @functools.partial(
      pl.kernel,
      out_type=jax.ShapeDtypeStruct((NC, NP, D), jnp.float32),
      mesh=mesh,
      scratch_types=[
          pltpu.VMEM((NCHUNK, CH), jnp.int32),
          pltpu.VMEM((CH, D), jnp.float32),
          pltpu.VMEM_SHARED((NP, D), jnp.float32),
      ],
  )
  def _cnt(dst_hbm, ones_hbm, zero_hbm, out_hbm, dst_v, ones_v, acc_sh):
    c = lax.axis_index("c")
    s = lax.axis_index("s")
    w = c * NS + s
    pltpu.sync_copy(dst_hbm.at[w], dst_v)
    pltpu.sync_copy(ones_hbm, ones_v)
    pltpu.sync_copy(zero_hbm.at[pl.ds(s * RPT, RPT)],
                    acc_sh.at[pl.ds(s * RPT, RPT)])
    plsc.subcore_barrier()

    def body(j, carry):
      pltpu.sync_copy(ones_v, acc_sh.at[dst_v.at[j]], add=True)
      return carry

    lax.fori_loop(0, NCHUNK, body, 0)
    plsc.subcore_barrier()
    pltpu.sync_copy(acc_sh.at[pl.ds(s * RPT, RPT)],
                    out_hbm.at[c].at[pl.ds(s * RPT, RPT)])

  return _cnt


BT = 1024


def _inv_body(c0_ref, c1_ref, o_ref):
    o_ref[...] = 1.0 / jnp.maximum(c0_ref[...] + c1_ref[...], 1.0)


def _inv(c0, c1):
    return pl.pallas_call(
        _inv_body,
        out_shape=jax.ShapeDtypeStruct((NP, D), jnp.float32),
        grid=(NP // BT,),
        in_specs=[
            pl.BlockSpec((BT, D), lambda i: (i, 0)),
            pl.BlockSpec((BT, D), lambda i: (i, 0)),
        ],
        out_specs=pl.BlockSpec((BT, D), lambda i: (i, 0)),
    )(c0, c1)


def _layer_body(p0_ref, p1_ref, inv_ref, x_ref, wl_ref, bl_ref, wr_ref, o_ref,
                *, relu):
    mean = (p0_ref[...] + p1_ref[...]) * inv_ref[...]
    h = jnp.dot(mean, wl_ref[...], preferred_element_type=jnp.float32)
    h = h + bl_ref[...]
    h = h + jnp.dot(x_ref[...], wr_ref[...], preferred_element_type=jnp.float32)
    if relu:
        h = jnp.maximum(h, 0.0)
    o_ref[...] = h


def _layer(p0, p1, inv, x, Wl, bl, Wr, relu):
    return pl.pallas_call(
        functools.partial(_layer_body, relu=relu),
        out_shape=jax.ShapeDtypeStruct((NP, D), jnp.float32),
        grid=(NP // BT,),
        in_specs=[
            pl.BlockSpec((BT, D), lambda i: (i, 0)),
            pl.BlockSpec((BT, D), lambda i: (i, 0)),
            pl.BlockSpec((BT, D), lambda i: (i, 0)),
            pl.BlockSpec((BT, D), lambda i: (i, 0)),
            pl.BlockSpec((D, D), lambda i: (0, 0)),
            pl.BlockSpec((1, D), lambda i: (0, 0)),
            pl.BlockSpec((D, D), lambda i: (0, 0)),
        ],
        out_specs=pl.BlockSpec((BT, D), lambda i: (i, 0)),
    )(p0, p1, inv, x, Wl, bl.reshape(1, D), Wr)


def kernel(x, edge_index, Wl0, bl0, Wr0, Wl1, bl1, Wr1, Wl2, bl2, Wr2):
    src = edge_index[0].reshape(NW, EPW)
    dst = edge_index[1].reshape(NW, NCHUNK, CH)
    zeros = jnp.zeros((NP, D), jnp.float32)
    ones = jnp.ones((CH, D), jnp.float32)

    cnt = _make_cnt()(dst, ones, zeros)
    inv = _inv(cnt[0], cnt[1])

    agg = _make_agg()
    h = jnp.pad(x, ((0, NP - N), (0, 0)))
    for Wl, bl, Wr, relu in ((Wl0, bl0, Wr0, True),
                             (Wl1, bl1, Wr1, True),
                             (Wl2, bl2, Wr2, False)):
        p = agg(h, src, dst, zeros)
        h = _layer(p[0], p[1], inv, h, Wl, bl, Wr, relu)
    return h[:N]


# speedup vs baseline: 9.7136x; 1.1585x over previous
import functools

import jax
import jax.numpy as jnp
from jax import lax
from jax.experimental import pallas as pl
from jax.experimental.pallas import tpu as pltpu
from jax.experimental.pallas import tpu_sc as plsc

N = 10000
NP = 10240
E = 320000
D = 128
NC = 2
NS = 16
NW = NC * NS
EPW = E // NW
CH = 80
NCHUNK = EPW // CH
RPT = NP // NS

@functools.cache
def _make_agg():
  mesh = plsc.VectorSubcoreMesh(core_axis_name="c", subcore_axis_name="s")

  @functools.partial(
      pl.kernel,
      out_type=jax.ShapeDtypeStruct((NC, NP, D), jnp.float32),
      mesh=mesh,
      scratch_types=[
          pltpu.VMEM((EPW,), jnp.int32),
          pltpu.VMEM((NCHUNK, CH), jnp.int32),
          pltpu.VMEM((CH, D), jnp.float32),
          pltpu.VMEM((CH, D), jnp.float32),
          pltpu.VMEM_SHARED((NP, D), jnp.float32),
          pltpu.SemaphoreType.DMA,
          pltpu.SemaphoreType.DMA,
          pltpu.SemaphoreType.DMA,
          pltpu.SemaphoreType.DMA,
      ],
  )
  def _agg(x_hbm, src_hbm, dst_hbm, zero_hbm, out_hbm, src_v, dst_v, rows_a,
           rows_b, acc_sh, sem_ga, sem_gb, sem_sa, sem_sb):
    c = lax.axis_index("c")
    s = lax.axis_index("s")
    w = c * NS + s
    pltpu.sync_copy(src_hbm.at[w], src_v)
    pltpu.sync_copy(dst_hbm.at[w], dst_v)
    pltpu.sync_copy(zero_hbm.at[pl.ds(s * RPT, RPT)],
                    acc_sh.at[pl.ds(s * RPT, RPT)])
    plsc.subcore_barrier()

    HC = CH // 2

    def fire_gather(j, buf, sem):
      pltpu.async_copy(x_hbm.at[src_v.at[pl.ds(j * CH, HC)]],
                       buf.at[pl.ds(0, HC)], sem)
      pltpu.async_copy(x_hbm.at[src_v.at[pl.ds(j * CH + HC, HC)]],
                       buf.at[pl.ds(HC, HC)], sem)

    def wait_gather(j, buf, sem):
      pltpu.make_async_copy(x_hbm.at[src_v.at[pl.ds(j * CH, HC)]],
                            buf.at[pl.ds(0, HC)], sem).wait()
      pltpu.make_async_copy(x_hbm.at[src_v.at[pl.ds(j * CH + HC, HC)]],
                            buf.at[pl.ds(HC, HC)], sem).wait()

    fire_gather(0, rows_a, sem_ga)
    fire_gather(1, rows_b, sem_gb)

    def body(i, carry):
      j0 = 2 * i
      wait_gather(j0, rows_a, sem_ga)
      pltpu.async_copy(rows_a, acc_sh.at[dst_v.at[j0]], sem_sa, add=True)
      wait_gather(j0 + 1, rows_b, sem_gb)
      pltpu.async_copy(rows_b, acc_sh.at[dst_v.at[j0 + 1]], sem_sb, add=True)

      @pl.when(j0 + 2 < NCHUNK)
      def _():
        pltpu.make_async_copy(rows_a, acc_sh.at[dst_v.at[j0]], sem_sa).wait()
        fire_gather(j0 + 2, rows_a, sem_ga)

      @pl.when(j0 + 3 < NCHUNK)
      def _():
        pltpu.make_async_copy(rows_b, acc_sh.at[dst_v.at[j0 + 1]],
                              sem_sb).wait()
        fire_gather(j0 + 3, rows_b, sem_gb)

      return carry

    lax.fori_loop(0, NCHUNK // 2, body, 0)
    wait_gather(NCHUNK - 1, rows_a, sem_ga)
    pltpu.async_copy(rows_a, acc_sh.at[dst_v.at[NCHUNK - 1]], sem_sa, add=True)
    pltpu.make_async_copy(rows_a, acc_sh.at[dst_v.at[NCHUNK - 1]],
                          sem_sa).wait()
    pltpu.make_async_copy(rows_b, acc_sh.at[dst_v.at[NCHUNK - 2]],
                          sem_sb).wait()
    plsc.subcore_barrier()
    pltpu.sync_copy(acc_sh.at[pl.ds(s * RPT, RPT)],
                    out_hbm.at[c].at[pl.ds(s * RPT, RPT)])

  return _agg


@functools.cache
def _make_hist():
  mesh = plsc.VectorSubcoreMesh(core_axis_name="c", subcore_axis_name="s")
  NBR = NP // D

  @functools.partial(
      pl.kernel,
      out_type=jax.ShapeDtypeStruct((NC, NBR, D), jnp.float32),
      mesh=mesh,
      compiler_params=pltpu.CompilerParams(needs_layout_passes=False),
      scratch_types=[
          pltpu.VMEM((NCHUNK, CH), jnp.int32),
          pltpu.VMEM((NBR, D), jnp.float32),
          pltpu.VMEM((NBR,), jnp.int32),
          pltpu.VMEM_SHARED((NBR, D), jnp.float32),
      ],
  )
  def _hist(dst_hbm, zero_hbm, out_hbm, dst_v, h_v, rows_v, acc_sh):
    c = lax.axis_index("c")
    s = lax.axis_index("s")
    w = c * NS + s
    pltpu.sync_copy(dst_hbm.at[w], dst_v)
    pltpu.sync_copy(zero_hbm.at[pl.ds(0, NBR)], h_v)

    @pl.when(s == 0)
    def _():
      pltpu.sync_copy(zero_hbm.at[pl.ds(0, NBR)], acc_sh)

    def iotas(k, carry):
      rows_v[pl.ds(k * 16, 16)] = lax.iota(jnp.int32, 16) + k * 16
      return carry

    lax.fori_loop(0, NBR // 16, iotas, 0)
    ones = jnp.ones((16,), jnp.float32)

    def body(i, carry):
      r = i // (CH // 16)
      k = i % (CH // 16)
      ix = dst_v[r, pl.ds(k * 16, 16)]
      plsc.addupdate_scatter(h_v, [ix >> 7, ix & 127], ones)
      return carry

    lax.fori_loop(0, NCHUNK * (CH // 16), body, 0)
    plsc.subcore_barrier()
    pltpu.sync_copy(h_v, acc_sh.at[rows_v], add=True)
    plsc.subcore_barrier()

    @pl.when(s == 0)
    def _():
      pltpu.sync_copy(acc_sh, out_hbm.at[c])

  return _hist


BT = 1024


def _layer_body(p_ref, cb_ref, x_ref, wl_ref, bl_ref, wr_ref, o_ref, *, relu):
    nbr = BT // D
    cnt = cb_ref[0] + cb_ref[1]
    tiled = jnp.tile(cnt.T, (nbr, 1))
    rsel = jax.lax.broadcasted_iota(jnp.int32, (BT, nbr), 0) // D
    csel = jax.lax.broadcasted_iota(jnp.int32, (BT, nbr), 1)
    cnt_col = jnp.sum(jnp.where(rsel == csel, tiled, 0.0), axis=1,
                      keepdims=True)
    inv = 1.0 / jnp.maximum(cnt_col, 1.0)
    mean = (p_ref[0] + p_ref[1]) * inv
    h = jnp.dot(mean, wl_ref[...], preferred_element_type=jnp.float32)
    h = h + bl_ref[...]
    h = h + jnp.dot(x_ref[...], wr_ref[...], preferred_element_type=jnp.float32)
    if relu:
        h = jnp.maximum(h, 0.0)
    o_ref[...] = h


def _layer(p, cb, x, Wl, bl, Wr, relu):
    xr = x.shape[0]
    return pl.pallas_call(
        functools.partial(_layer_body, relu=relu),
        out_shape=jax.ShapeDtypeStruct((NP, D), jnp.float32),
        grid=(NP // BT,),
        in_specs=[
            pl.BlockSpec((NC, BT, D), lambda i: (0, i, 0)),
            pl.BlockSpec((NC, BT // D, D), lambda i: (0, i, 0)),
            pl.BlockSpec((BT, D), lambda i: (i, 0)),
            pl.BlockSpec((D, D), lambda i: (0, 0)),
            pl.BlockSpec((1, D), lambda i: (0, 0)),
            pl.BlockSpec((D, D), lambda i: (0, 0)),
        ],
        out_specs=pl.BlockSpec((BT, D), lambda i: (i, 0)),
    )(p, cb, x, Wl, bl.reshape(1, D), Wr)


def kernel(x, edge_index, Wl0, bl0, Wr0, Wl1, bl1, Wr1, Wl2, bl2, Wr2):
    src = edge_index[0].reshape(NW, EPW)
    dst = edge_index[1].reshape(NW, NCHUNK, CH)
    zeros = jnp.zeros((NP, D), jnp.float32)

    cb = _make_hist()(dst, zeros)

    agg = _make_agg()
    h = jnp.pad(x, ((0, NP - N), (0, 0)))
    for Wl, bl, Wr, relu in ((Wl0, bl0, Wr0, True),
                             (Wl1, bl1, Wr1, True),
                             (Wl2, bl2, Wr2, False)):
        p = agg(h, src, dst, zeros)
        h = _layer(p, cb, h, Wl, bl, Wr, relu)
    return h[:N]


# speedup vs baseline: 9.8306x; 1.0120x over previous
import functools

import jax
import jax.numpy as jnp
from jax import lax
from jax.experimental import pallas as pl
from jax.experimental.pallas import tpu as pltpu
from jax.experimental.pallas import tpu_sc as plsc

N = 10000
NP = 10240
E = 320000
D = 128
NC = 2
NS = 16
NW = NC * NS
EPW = E // NW
CH = 80
NCHUNK = EPW // CH
RPT = NP // NS

@functools.cache
def _make_agg(xr):
  mesh = plsc.VectorSubcoreMesh(core_axis_name="c", subcore_axis_name="s")

  @functools.partial(
      pl.kernel,
      out_type=jax.ShapeDtypeStruct((NC, NP, D), jnp.float32),
      mesh=mesh,
      scratch_types=[
          pltpu.VMEM((EPW,), jnp.int32),
          pltpu.VMEM((NCHUNK, CH), jnp.int32),
          pltpu.VMEM((CH, D), jnp.float32),
          pltpu.VMEM((CH, D), jnp.float32),
          pltpu.VMEM_SHARED((NP, D), jnp.float32),
          pltpu.SemaphoreType.DMA,
          pltpu.SemaphoreType.DMA,
          pltpu.SemaphoreType.DMA,
          pltpu.SemaphoreType.DMA,
      ],
  )
  def _agg(x_hbm, src_hbm, dst_hbm, zero_hbm, out_hbm, src_v, dst_v, rows_a,
           rows_b, acc_sh, sem_ga, sem_gb, sem_sa, sem_sb):
    c = lax.axis_index("c")
    s = lax.axis_index("s")
    w = c * NS + s
    pltpu.sync_copy(src_hbm.at[w], src_v)
    pltpu.sync_copy(dst_hbm.at[w], dst_v)
    @pl.when(s < NS - 1)
    def _():
      pltpu.sync_copy(zero_hbm.at[pl.ds(s * 632, 632)],
                      acc_sh.at[pl.ds(s * 632, 632)])

    @pl.when(s == NS - 1)
    def _():
      pltpu.sync_copy(zero_hbm.at[pl.ds(15 * 632, 520)],
                      acc_sh.at[pl.ds(15 * 632, 520)])

    plsc.subcore_barrier()

    HC = CH // 2

    def fire_gather(j, buf, sem):
      pltpu.async_copy(x_hbm.at[src_v.at[pl.ds(j * CH, HC)]],
                       buf.at[pl.ds(0, HC)], sem)
      pltpu.async_copy(x_hbm.at[src_v.at[pl.ds(j * CH + HC, HC)]],
                       buf.at[pl.ds(HC, HC)], sem)

    def wait_gather(j, buf, sem):
      pltpu.make_async_copy(x_hbm.at[src_v.at[pl.ds(j * CH, HC)]],
                            buf.at[pl.ds(0, HC)], sem).wait()
      pltpu.make_async_copy(x_hbm.at[src_v.at[pl.ds(j * CH + HC, HC)]],
                            buf.at[pl.ds(HC, HC)], sem).wait()

    fire_gather(0, rows_a, sem_ga)
    fire_gather(1, rows_b, sem_gb)

    def body(i, carry):
      j0 = 2 * i
      wait_gather(j0, rows_a, sem_ga)
      pltpu.async_copy(rows_a, acc_sh.at[dst_v.at[j0]], sem_sa, add=True)
      wait_gather(j0 + 1, rows_b, sem_gb)
      pltpu.async_copy(rows_b, acc_sh.at[dst_v.at[j0 + 1]], sem_sb, add=True)

      @pl.when(j0 + 2 < NCHUNK)
      def _():
        pltpu.make_async_copy(rows_a, acc_sh.at[dst_v.at[j0]], sem_sa).wait()
        fire_gather(j0 + 2, rows_a, sem_ga)

      @pl.when(j0 + 3 < NCHUNK)
      def _():
        pltpu.make_async_copy(rows_b, acc_sh.at[dst_v.at[j0 + 1]],
                              sem_sb).wait()
        fire_gather(j0 + 3, rows_b, sem_gb)

      return carry

    lax.fori_loop(0, NCHUNK // 2, body, 0)
    wait_gather(NCHUNK - 1, rows_a, sem_ga)
    pltpu.async_copy(rows_a, acc_sh.at[dst_v.at[NCHUNK - 1]], sem_sa, add=True)
    pltpu.make_async_copy(rows_a, acc_sh.at[dst_v.at[NCHUNK - 1]],
                          sem_sa).wait()
    pltpu.make_async_copy(rows_b, acc_sh.at[dst_v.at[NCHUNK - 2]],
                          sem_sb).wait()
    plsc.subcore_barrier()

    @pl.when(s < NS - 1)
    def _():
      pltpu.sync_copy(acc_sh.at[pl.ds(s * 632, 632)],
                      out_hbm.at[c].at[pl.ds(s * 632, 632)])

    @pl.when(s == NS - 1)
    def _():
      pltpu.sync_copy(acc_sh.at[pl.ds(15 * 632, 520)],
                      out_hbm.at[c].at[pl.ds(15 * 632, 520)])

  return _agg


@functools.cache
def _make_hist():
  mesh = plsc.VectorSubcoreMesh(core_axis_name="c", subcore_axis_name="s")
  NBR = NP // D

  @functools.partial(
      pl.kernel,
      out_type=jax.ShapeDtypeStruct((NC, NBR, D), jnp.float32),
      mesh=mesh,
      compiler_params=pltpu.CompilerParams(needs_layout_passes=False),
      scratch_types=[
          pltpu.VMEM((NCHUNK, CH), jnp.int32),
          pltpu.VMEM((NBR, D), jnp.float32),
          pltpu.VMEM((NBR,), jnp.int32),
          pltpu.VMEM_SHARED((NBR, D), jnp.float32),
      ],
  )
  def _hist(dst_hbm, zero_hbm, out_hbm, dst_v, h_v, rows_v, acc_sh):
    c = lax.axis_index("c")
    s = lax.axis_index("s")
    w = c * NS + s
    pltpu.sync_copy(dst_hbm.at[w], dst_v)
    pltpu.sync_copy(zero_hbm.at[pl.ds(0, NBR)], h_v)

    @pl.when(s == 0)
    def _():
      pltpu.sync_copy(zero_hbm.at[pl.ds(0, NBR)], acc_sh)

    def iotas(k, carry):
      rows_v[pl.ds(k * 16, 16)] = lax.iota(jnp.int32, 16) + k * 16
      return carry

    lax.fori_loop(0, NBR // 16, iotas, 0)
    ones = jnp.ones((16,), jnp.float32)

    def body(i, carry):
      r = i // (CH // 16)
      k = i % (CH // 16)
      ix = dst_v[r, pl.ds(k * 16, 16)]
      plsc.addupdate_scatter(h_v, [ix >> 7, ix & 127], ones)
      return carry

    lax.fori_loop(0, NCHUNK * (CH // 16), body, 0)
    plsc.subcore_barrier()
    pltpu.sync_copy(h_v, acc_sh.at[rows_v], add=True)
    plsc.subcore_barrier()

    @pl.when(s == 0)
    def _():
      pltpu.sync_copy(acc_sh, out_hbm.at[c])

  return _hist


BT = 1024


def _layer_body(p_ref, cb_ref, x_ref, wl_ref, bl_ref, wr_ref, o_ref, *, relu):
    nbr = BT // D
    cnt = cb_ref[0] + cb_ref[1]
    tiled = jnp.tile(cnt.T, (nbr, 1))
    rsel = jax.lax.broadcasted_iota(jnp.int32, (BT, nbr), 0) // D
    csel = jax.lax.broadcasted_iota(jnp.int32, (BT, nbr), 1)
    cnt_col = jnp.sum(jnp.where(rsel == csel, tiled, 0.0), axis=1,
                      keepdims=True)
    inv = 1.0 / jnp.maximum(cnt_col, 1.0)
    mean = (p_ref[0] + p_ref[1]) * inv
    h = jnp.dot(mean, wl_ref[...], preferred_element_type=jnp.float32)
    h = h + bl_ref[...]
    h = h + jnp.dot(x_ref[...], wr_ref[...], preferred_element_type=jnp.float32)
    if relu:
        h = jnp.maximum(h, 0.0)
    o_ref[...] = h


def _layer(p, cb, x, Wl, bl, Wr, relu, out_rows=NP):
    return pl.pallas_call(
        functools.partial(_layer_body, relu=relu),
        out_shape=jax.ShapeDtypeStruct((out_rows, D), jnp.float32),
        grid=(NP // BT,),
        in_specs=[
            pl.BlockSpec((NC, BT, D), lambda i: (0, i, 0)),
            pl.BlockSpec((NC, BT // D, D), lambda i: (0, i, 0)),
            pl.BlockSpec((BT, D), lambda i: (i, 0)),
            pl.BlockSpec((D, D), lambda i: (0, 0)),
            pl.BlockSpec((1, D), lambda i: (0, 0)),
            pl.BlockSpec((D, D), lambda i: (0, 0)),
        ],
        out_specs=pl.BlockSpec((BT, D), lambda i: (i, 0)),
    )(p, cb, x, Wl, bl.reshape(1, D), Wr)


def kernel(x, edge_index, Wl0, bl0, Wr0, Wl1, bl1, Wr1, Wl2, bl2, Wr2):
    src = edge_index[0].reshape(NW, EPW)
    dst = edge_index[1].reshape(NW, NCHUNK, CH)
    zeros = jnp.zeros((NP, D), jnp.float32)

    cb = _make_hist()(dst, zeros)

    h = x
    for li, (Wl, bl, Wr, relu) in enumerate(((Wl0, bl0, Wr0, True),
                                             (Wl1, bl1, Wr1, True),
                                             (Wl2, bl2, Wr2, False))):
        p = _make_agg(h.shape[0])(h, src, dst, zeros)
        h = _layer(p, cb, h, Wl, bl, Wr, relu,
                   out_rows=N if li == 2 else NP)
    return h


# speedup vs baseline: 10.1029x; 1.0277x over previous
import functools

import jax
import jax.numpy as jnp
from jax import lax
from jax.experimental import pallas as pl
from jax.experimental.pallas import tpu as pltpu
from jax.experimental.pallas import tpu_sc as plsc

N = 10000
NP = 10240
E = 320000
D = 128
NC = 2
NS = 16
NW = NC * NS
EPW = E // NW
CH = 80
NCHUNK = EPW // CH
RPT = NP // NS

@functools.cache
def _make_agg(xr):
  mesh = plsc.VectorSubcoreMesh(core_axis_name="c", subcore_axis_name="s")

  @functools.partial(
      pl.kernel,
      out_type=jax.ShapeDtypeStruct((NC, NP, D), jnp.float32),
      mesh=mesh,
      scratch_types=[
          pltpu.VMEM((EPW,), jnp.int32),
          pltpu.VMEM((EPW,), jnp.int32),
          pltpu.VMEM((CH, D), jnp.float32),
          pltpu.VMEM((CH, D), jnp.float32),
          pltpu.VMEM_SHARED((NP, D), jnp.float32),
          pltpu.SemaphoreType.DMA,
          pltpu.SemaphoreType.DMA,
          pltpu.SemaphoreType.DMA,
          pltpu.SemaphoreType.DMA,
      ],
  )
  def _agg(x_hbm, ei_hbm, zero_hbm, out_hbm, src_v, dst_v, rows_a,
           rows_b, acc_sh, sem_ga, sem_gb, sem_sa, sem_sb):
    c = lax.axis_index("c")
    s = lax.axis_index("s")
    w = c * NS + s
    pltpu.sync_copy(ei_hbm.at[0, w], src_v)
    pltpu.sync_copy(ei_hbm.at[1, w], dst_v)
    @pl.when(s < NS - 1)
    def _():
      pltpu.sync_copy(zero_hbm.at[pl.ds(s * 632, 632)],
                      acc_sh.at[pl.ds(s * 632, 632)])

    @pl.when(s == NS - 1)
    def _():
      pltpu.sync_copy(zero_hbm.at[pl.ds(15 * 632, 520)],
                      acc_sh.at[pl.ds(15 * 632, 520)])

    plsc.subcore_barrier()

    HC = CH // 2

    def fire_gather(j, buf, sem):
      pltpu.async_copy(x_hbm.at[src_v.at[pl.ds(j * CH, HC)]],
                       buf.at[pl.ds(0, HC)], sem)
      pltpu.async_copy(x_hbm.at[src_v.at[pl.ds(j * CH + HC, HC)]],
                       buf.at[pl.ds(HC, HC)], sem)

    def wait_gather(j, buf, sem):
      pltpu.make_async_copy(x_hbm.at[src_v.at[pl.ds(j * CH, HC)]],
                            buf.at[pl.ds(0, HC)], sem).wait()
      pltpu.make_async_copy(x_hbm.at[src_v.at[pl.ds(j * CH + HC, HC)]],
                            buf.at[pl.ds(HC, HC)], sem).wait()

    fire_gather(0, rows_a, sem_ga)
    fire_gather(1, rows_b, sem_gb)

    def body(i, carry):
      j0 = 2 * i
      wait_gather(j0, rows_a, sem_ga)
      pltpu.async_copy(rows_a, acc_sh.at[dst_v.at[pl.ds(j0 * CH, CH)]], sem_sa, add=True)
      wait_gather(j0 + 1, rows_b, sem_gb)
      pltpu.async_copy(rows_b, acc_sh.at[dst_v.at[pl.ds((j0 + 1) * CH, CH)]], sem_sb, add=True)

      @pl.when(j0 + 2 < NCHUNK)
      def _():
        pltpu.make_async_copy(rows_a, acc_sh.at[dst_v.at[pl.ds(j0 * CH, CH)]], sem_sa).wait()
        fire_gather(j0 + 2, rows_a, sem_ga)

      @pl.when(j0 + 3 < NCHUNK)
      def _():
        pltpu.make_async_copy(rows_b, acc_sh.at[dst_v.at[pl.ds((j0 + 1) * CH, CH)]],
                              sem_sb).wait()
        fire_gather(j0 + 3, rows_b, sem_gb)

      return carry

    lax.fori_loop(0, NCHUNK // 2, body, 0)
    wait_gather(NCHUNK - 1, rows_a, sem_ga)
    pltpu.async_copy(rows_a, acc_sh.at[dst_v.at[pl.ds((NCHUNK - 1) * CH, CH)]], sem_sa, add=True)
    pltpu.make_async_copy(rows_a, acc_sh.at[dst_v.at[pl.ds((NCHUNK - 1) * CH, CH)]],
                          sem_sa).wait()
    pltpu.make_async_copy(rows_b, acc_sh.at[dst_v.at[pl.ds((NCHUNK - 2) * CH, CH)]],
                          sem_sb).wait()
    plsc.subcore_barrier()

    @pl.when(s < NS - 1)
    def _():
      pltpu.sync_copy(acc_sh.at[pl.ds(s * 632, 632)],
                      out_hbm.at[c].at[pl.ds(s * 632, 632)])

    @pl.when(s == NS - 1)
    def _():
      pltpu.sync_copy(acc_sh.at[pl.ds(15 * 632, 520)],
                      out_hbm.at[c].at[pl.ds(15 * 632, 520)])

  return _agg


@functools.cache
def _make_hist():
  mesh = plsc.VectorSubcoreMesh(core_axis_name="c", subcore_axis_name="s")
  NBR = NP // D

  @functools.partial(
      pl.kernel,
      out_type=jax.ShapeDtypeStruct((NC, NBR, D), jnp.float32),
      mesh=mesh,
      compiler_params=pltpu.CompilerParams(needs_layout_passes=False),
      scratch_types=[
          pltpu.VMEM((EPW,), jnp.int32),
          pltpu.VMEM((NBR, D), jnp.float32),
          pltpu.VMEM((NBR,), jnp.int32),
          pltpu.VMEM_SHARED((NBR, D), jnp.float32),
      ],
  )
  def _hist(ei_hbm, zero_hbm, out_hbm, dst_v, h_v, rows_v, acc_sh):
    c = lax.axis_index("c")
    s = lax.axis_index("s")
    w = c * NS + s
    pltpu.sync_copy(ei_hbm.at[1, w], dst_v)
    pltpu.sync_copy(zero_hbm.at[pl.ds(0, NBR)], h_v)

    @pl.when(s == 0)
    def _():
      pltpu.sync_copy(zero_hbm.at[pl.ds(0, NBR)], acc_sh)

    def iotas(k, carry):
      rows_v[pl.ds(k * 16, 16)] = lax.iota(jnp.int32, 16) + k * 16
      return carry

    lax.fori_loop(0, NBR // 16, iotas, 0)
    ones = jnp.ones((16,), jnp.float32)

    def body(i, carry):
      ix = dst_v[pl.ds(i * 16, 16)]
      plsc.addupdate_scatter(h_v, [ix >> 7, ix & 127], ones)
      return carry

    lax.fori_loop(0, EPW // 16, body, 0)
    plsc.subcore_barrier()
    pltpu.sync_copy(h_v, acc_sh.at[rows_v], add=True)
    plsc.subcore_barrier()

    @pl.when(s == 0)
    def _():
      pltpu.sync_copy(acc_sh, out_hbm.at[c])

  return _hist


BT = 1024


def _layer_body(p_ref, cb_ref, x_ref, wl_ref, bl_ref, wr_ref, o_ref, *, relu):
    nbr = BT // D
    cnt = cb_ref[0] + cb_ref[1]
    tiled = jnp.tile(cnt.T, (nbr, 1))
    rsel = jax.lax.broadcasted_iota(jnp.int32, (BT, nbr), 0) // D
    csel = jax.lax.broadcasted_iota(jnp.int32, (BT, nbr), 1)
    cnt_col = jnp.sum(jnp.where(rsel == csel, tiled, 0.0), axis=1,
                      keepdims=True)
    inv = 1.0 / jnp.maximum(cnt_col, 1.0)
    mean = (p_ref[0] + p_ref[1]) * inv
    h = jnp.dot(mean, wl_ref[...], preferred_element_type=jnp.float32)
    h = h + bl_ref[...]
    h = h + jnp.dot(x_ref[...], wr_ref[...], preferred_element_type=jnp.float32)
    if relu:
        h = jnp.maximum(h, 0.0)
    o_ref[...] = h


def _layer(p, cb, x, Wl, bl, Wr, relu, out_rows=NP):
    return pl.pallas_call(
        functools.partial(_layer_body, relu=relu),
        out_shape=jax.ShapeDtypeStruct((out_rows, D), jnp.float32),
        grid=(NP // BT,),
        in_specs=[
            pl.BlockSpec((NC, BT, D), lambda i: (0, i, 0)),
            pl.BlockSpec((NC, BT // D, D), lambda i: (0, i, 0)),
            pl.BlockSpec((BT, D), lambda i: (i, 0)),
            pl.BlockSpec((D, D), lambda i: (0, 0)),
            pl.BlockSpec((1, D), lambda i: (0, 0)),
            pl.BlockSpec((D, D), lambda i: (0, 0)),
        ],
        out_specs=pl.BlockSpec((BT, D), lambda i: (i, 0)),
    )(p, cb, x, Wl, bl.reshape(1, D), Wr)


def kernel(x, edge_index, Wl0, bl0, Wr0, Wl1, bl1, Wr1, Wl2, bl2, Wr2):
    ei = edge_index.reshape(2, NW, EPW)
    zeros = jnp.zeros((NP, D), jnp.float32)

    cb = _make_hist()(ei, zeros)

    h = x
    for li, (Wl, bl, Wr, relu) in enumerate(((Wl0, bl0, Wr0, True),
                                             (Wl1, bl1, Wr1, True),
                                             (Wl2, bl2, Wr2, False))):
        p = _make_agg(h.shape[0])(h, ei, zeros)
        h = _layer(p, cb, h, Wl, bl, Wr, relu,
                   out_rows=N if li == 2 else NP)
    return h
